# baseline (speedup 1.0000x reference)
import jax
import jax.numpy as jnp
from jax import lax
from jax.experimental import pallas as pl


def _cox_body(t_col, t_row, p_col, p_row, ind_col, out_acc):
    i = pl.program_id(0)
    mask = (t_row[...] >= t_col[...]).astype(jnp.float32)
    e_row = jnp.exp(p_row[...])
    s = lax.dot_general(mask, e_row, (((1,), (1,)), ((), ())),
                        preferred_element_type=jnp.float32)
    diffs = p_col[...] - jnp.log(s)
    ind = ind_col[...]
    num = jnp.sum(ind * diffs)
    den = jnp.sum(ind)
    vals = jnp.concatenate(
        [num.reshape(1, 1), den.reshape(1, 1)], axis=1)

    @pl.when(i == 0)
    def _init():
        out_acc[...] = jnp.zeros_like(out_acc)

    out_acc[...] += vals


@jax.jit
def kernel(pred, gt_indicator, gt_time):
    B = pred.shape[0]
    R = 256
    t_col = gt_time.reshape(B, 1)
    t_row = gt_time.reshape(1, B)
    p_col = pred.reshape(B, 1)
    p_row = pred.reshape(1, B)
    ind_col = gt_indicator.astype(jnp.float32).reshape(B, 1)

    acc = pl.pallas_call(
        _cox_body,
        grid=(B // R,),
        in_specs=[
            pl.BlockSpec((R, 1), lambda i: (i, 0)),
            pl.BlockSpec((1, B), lambda i: (0, 0)),
            pl.BlockSpec((R, 1), lambda i: (i, 0)),
            pl.BlockSpec((1, B), lambda i: (0, 0)),
            pl.BlockSpec((R, 1), lambda i: (i, 0)),
        ],
        out_specs=pl.BlockSpec((1, 2), lambda i: (0, 0)),
        out_shape=jax.ShapeDtypeStruct((1, 2), jnp.float32),
    )(t_col, t_row, p_col, p_row, ind_col)

    return -(acc[0, 0] / acc[0, 1])


# speedup vs baseline: 1.5129x; 1.5129x over previous
import jax
import jax.numpy as jnp
from jax import lax
from jax.experimental import pallas as pl
from jax.experimental.pallas import tpu as pltpu

UNROLL = 8


def _cox_body(t_s, e_s, t2, p2, ind2, out_acc):
    B = t_s.shape[1]
    trows = t2[...]
    acc0 = jnp.zeros_like(trows)

    def body(it, acc):
        j = it * UNROLL
        for u in range(UNROLL):
            tj = t_s[0, j + u]
            ej = e_s[0, j + u]
            acc = acc + jnp.where(tj >= trows, ej, jnp.float32(0.0))
        return acc

    s = lax.fori_loop(0, B // UNROLL, body, acc0)
    ind = ind2[...]
    diffs = p2[...] - jnp.log(s)
    num = jnp.sum(ind * diffs)
    den = jnp.sum(ind)
    out_acc[...] = jnp.concatenate(
        [num.reshape(1, 1), den.reshape(1, 1)], axis=1)


@jax.jit
def kernel(pred, gt_indicator, gt_time):
    B = pred.shape[0]
    t_s = gt_time.reshape(1, B)
    e_s = jnp.exp(pred).reshape(1, B)
    t2 = gt_time.reshape(32, 128)
    p2 = pred.reshape(32, 128)
    ind2 = gt_indicator.astype(jnp.float32).reshape(32, 128)

    acc = pl.pallas_call(
        _cox_body,
        in_specs=[
            pl.BlockSpec(memory_space=pltpu.SMEM),
            pl.BlockSpec(memory_space=pltpu.SMEM),
            pl.BlockSpec((32, 128), lambda: (0, 0)),
            pl.BlockSpec((32, 128), lambda: (0, 0)),
            pl.BlockSpec((32, 128), lambda: (0, 0)),
        ],
        out_specs=pl.BlockSpec((1, 2), lambda: (0, 0)),
        out_shape=jax.ShapeDtypeStruct((1, 2), jnp.float32),
    )(t_s, e_s, t2, p2, ind2)

    return -(acc[0, 0] / acc[0, 1])


# speedup vs baseline: 1.6119x; 1.0654x over previous
import jax
import jax.numpy as jnp
from jax import lax
from jax.experimental import pallas as pl
from jax.experimental.pallas import tpu as pltpu

UNROLL = 16
NACC = 4


def _cox_body(t_s, e_s, t2, p2, ind2, out_acc):
    B = t_s.shape[1]
    trows = t2[...]
    acc0 = tuple(jnp.zeros_like(trows) for _ in range(NACC))

    def body(it, accs):
        j = it * UNROLL
        accs = list(accs)
        for u in range(UNROLL):
            tj = t_s[0, j + u]
            ej = e_s[0, j + u]
            a = u % NACC
            accs[a] = accs[a] + jnp.where(tj >= trows, ej,
                                          jnp.float32(0.0))
        return tuple(accs)

    accs = lax.fori_loop(0, B // UNROLL, body, acc0)
    s = (accs[0] + accs[1]) + (accs[2] + accs[3])
    ind = ind2[...]
    diffs = p2[...] - jnp.log(s)
    num = jnp.sum(ind * diffs)
    den = jnp.sum(ind)
    out_acc[...] = jnp.concatenate(
        [num.reshape(1, 1), den.reshape(1, 1)], axis=1)


@jax.jit
def kernel(pred, gt_indicator, gt_time):
    B = pred.shape[0]
    t_s = gt_time.reshape(1, B)
    e_s = jnp.exp(pred).reshape(1, B)
    t2 = gt_time.reshape(32, 128)
    p2 = pred.reshape(32, 128)
    ind2 = gt_indicator.astype(jnp.float32).reshape(32, 128)

    acc = pl.pallas_call(
        _cox_body,
        in_specs=[
            pl.BlockSpec(memory_space=pltpu.SMEM),
            pl.BlockSpec(memory_space=pltpu.SMEM),
            pl.BlockSpec((32, 128), lambda: (0, 0)),
            pl.BlockSpec((32, 128), lambda: (0, 0)),
            pl.BlockSpec((32, 128), lambda: (0, 0)),
        ],
        out_specs=pl.BlockSpec((1, 2), lambda: (0, 0)),
        out_shape=jax.ShapeDtypeStruct((1, 2), jnp.float32),
    )(t_s, e_s, t2, p2, ind2)

    return -(acc[0, 0] / acc[0, 1])


# speedup vs baseline: 1.9982x; 1.2396x over previous
import jax
import jax.numpy as jnp
from jax import lax
from jax.experimental import pallas as pl
from jax.experimental.pallas import tpu as pltpu

UNROLL = 16
NACC = 4


def _cox_body(t_s, e_s, t2, p2, ind2, out_ref):
    B = t_s.shape[1]
    trows = t2[...]
    acc0 = tuple(jnp.zeros_like(trows) for _ in range(NACC))

    def body(it, accs):
        j = it * UNROLL
        accs = list(accs)
        for u in range(UNROLL):
            tj = t_s[0, j + u]
            ej = e_s[0, j + u]
            a = u % NACC
            accs[a] = accs[a] + jnp.where(tj >= trows, ej,
                                          jnp.float32(0.0))
        return tuple(accs)

    accs = lax.fori_loop(0, B // UNROLL, body, acc0)
    s = (accs[0] + accs[1]) + (accs[2] + accs[3])
    ind = ind2[...].astype(jnp.float32)
    diffs = p2[...] - jnp.log(s)
    num = jnp.sum(ind * diffs)
    den = jnp.sum(ind)
    out_ref[...] = (-(num / den)).reshape(1, 1)


@jax.jit
def kernel(pred, gt_indicator, gt_time):
    B = pred.shape[0]
    t_s = gt_time.reshape(1, B)
    e_s = jnp.exp(pred).reshape(1, B)
    t2 = gt_time.reshape(32, 128)
    p2 = pred.reshape(32, 128)
    ind2 = gt_indicator.reshape(32, 128)

    out = pl.pallas_call(
        _cox_body,
        in_specs=[
            pl.BlockSpec(memory_space=pltpu.SMEM),
            pl.BlockSpec(memory_space=pltpu.SMEM),
            pl.BlockSpec((32, 128), lambda: (0, 0)),
            pl.BlockSpec((32, 128), lambda: (0, 0)),
            pl.BlockSpec((32, 128), lambda: (0, 0)),
        ],
        out_specs=pl.BlockSpec((1, 1), lambda: (0, 0)),
        out_shape=jax.ShapeDtypeStruct((1, 1), jnp.float32),
    )(t_s, e_s, t2, p2, ind2)

    return out.reshape(())
